# baseline (speedup 1.0000x reference)
import jax
import jax.numpy as jnp
from jax import lax
from jax.experimental import pallas as pl
from jax.experimental.pallas import tpu as pltpu
from jax.experimental.pallas import tpu_sc as plsc

_B, _C, _H, _W, _K = 16, 64, 128, 128, 128
_HW = _H * _W
_NC, _NS, _L = 2, 16, 16
_NW = _NC * _NS
_KCHUNK = (_B * _K) // _NW
_KG = _KCHUNK // _L
_NROW = (_KCHUNK * _C) // 128
_GROUP = 8


def _body(feat_hbm, tgt_hbm, ind_hbm, mask_hbm, out_hbm,
          ind_v, mask_v, idx_v, tgt_v, pred_v, out_v,
          sem0, sem1, sem2, sem3, sem_t, sem_i):
  wid = lax.axis_index("s") * _NC + lax.axis_index("c")
  b = wid // 2
  k0 = (wid % 2) * _KCHUNK

  cp_ind = pltpu.make_async_copy(
      ind_hbm.at[b, pl.ds(k0, _KCHUNK)], ind_v, sem_i)
  cp_ind.start()
  cp_tgt = pltpu.make_async_copy(
      tgt_hbm.at[b, pl.ds(k0, _KCHUNK), :], tgt_v, sem_t)
  cp_tgt.start()
  cp_mask = pltpu.make_async_copy(
      mask_hbm.at[b, pl.ds(k0, _KCHUNK)], mask_v, sem_t)
  cp_mask.start()
  cp_ind.wait()

  iota = lax.iota(jnp.int32, _L)
  base = b * _C * _HW

  def build(j, carry):
    for h in range(2):
      off = base + (2 * j + h) * _HW
      for kg in range(_KG):
        kv = ind_v[pl.ds(kg * _L, _L)]
        idx_v[j, pl.ds(h * 64 + kg * _L, _L)] = kv + off
    return carry

  sems = (sem0, sem1, sem2, sem3)
  ngroups = _NROW // _GROUP

  waves = [[pltpu.make_async_copy(feat_hbm.at[idx_v.at[g * _GROUP + i]],
                                  pred_v.at[g * _GROUP + i], sems[g])
            for i in range(_GROUP)]
           for g in range(ngroups)]
  for g in range(ngroups):
    lax.fori_loop(g * _GROUP, (g + 1) * _GROUP, build, 0)
    for cp in waves[g]:
      cp.start()

  cp_tgt.wait()
  cp_mask.wait()

  def cbody(j, accs):
    a = list(accs)
    for h in range(2):
      cc = 2 * j + h
      for kg in range(_KG):
        pv = pred_v[j, pl.ds(h * 64 + kg * _L, _L)]
        tv = plsc.load_gather(tgt_v, [iota + kg * _L,
                                      jnp.zeros((_L,), jnp.int32) + cc])
        a[kg] = a[kg] + jnp.abs(pv - tv)
    return tuple(a)

  z = jnp.zeros((_L,), jnp.float32)
  accs = (z,) * _KG
  for g in range(ngroups):
    for cp in waves[g]:
      cp.wait()
    accs = lax.fori_loop(g * _GROUP, (g + 1) * _GROUP, cbody, accs,
                         unroll=2)

  num = jnp.float32(0.0)
  den = jnp.float32(0.0)
  for kg in range(_KG):
    mv = mask_v[pl.ds(kg * _L, _L)]
    num = num + jnp.sum(accs[kg] * jnp.abs(mv))
    den = den + jnp.sum(mv)

  out_v[...] = jnp.where(iota == 0, num, jnp.where(iota == 1, den, 0.0))
  pltpu.sync_copy(out_v, out_hbm.at[wid])


_SC_LOSS = pl.kernel(
    _body,
    out_type=jax.ShapeDtypeStruct((_NW, _L), jnp.float32),
    mesh=plsc.VectorSubcoreMesh(core_axis_name="c", subcore_axis_name="s"),
    compiler_params=pltpu.CompilerParams(needs_layout_passes=False),
    scratch_types=[
        pltpu.VMEM((_KCHUNK,), jnp.int32),
        pltpu.VMEM((_KCHUNK,), jnp.float32),
        pltpu.VMEM((_NROW, 128), jnp.int32),
        pltpu.VMEM((_KCHUNK, _C), jnp.float32),
        pltpu.VMEM((_NROW, 128), jnp.float32),
        pltpu.VMEM((_L,), jnp.float32),
        pltpu.SemaphoreType.DMA,
        pltpu.SemaphoreType.DMA,
        pltpu.SemaphoreType.DMA,
        pltpu.SemaphoreType.DMA,
        pltpu.SemaphoreType.DMA,
        pltpu.SemaphoreType.DMA,
    ],
)


@jax.jit
def kernel(output, mask, ind, target, freq_mask, hm):
  feat = output.reshape(_B * _C * _HW)
  parts = _SC_LOSS(feat, target, ind, mask)
  tot = jnp.sum(parts, axis=0)
  return tot[0] / (_C * tot[1] + 1e-4)


# speedup vs baseline: 1.0074x; 1.0074x over previous
import jax
import jax.numpy as jnp
from jax import lax
from jax.experimental import pallas as pl
from jax.experimental.pallas import tpu as pltpu
from jax.experimental.pallas import tpu_sc as plsc

_B, _C, _H, _W, _K = 16, 64, 128, 128, 128
_HW = _H * _W
_NC, _NS, _L = 2, 16, 16
_NW = _NC * _NS
_KCHUNK = (_B * _K) // _NW
_KG = _KCHUNK // _L
_NROW = (_KCHUNK * _C) // 128
_GROUP = 8


def _body(feat_hbm, tgt_hbm, ind_hbm, mask_hbm, out_hbm,
          ind_v, mask_v, idx_v, tgt_v, pred_v, out_v,
          sem0, sem1, sem2, sem3, sem_t, sem_i):
  wid = lax.axis_index("s") * _NC + lax.axis_index("c")
  b = wid // 2
  k0 = (wid % 2) * _KCHUNK

  cp_ind = pltpu.make_async_copy(
      ind_hbm.at[b, pl.ds(k0, _KCHUNK)], ind_v, sem_i)
  cp_ind.start()
  cp_tgt = pltpu.make_async_copy(
      tgt_hbm.at[b, pl.ds(k0, _KCHUNK), :], tgt_v, sem_t)
  cp_tgt.start()
  cp_mask = pltpu.make_async_copy(
      mask_hbm.at[b, pl.ds(k0, _KCHUNK)], mask_v, sem_t)
  cp_mask.start()
  cp_ind.wait()

  iota = lax.iota(jnp.int32, _L)
  base = b * _C * _HW

  def build(j, carry):
    for h in range(2):
      off = base + (2 * j + h) * _HW
      for kg in range(_KG):
        kv = ind_v[pl.ds(kg * _L, _L)]
        idx_v[j, pl.ds(h * 64 + kg * _L, _L)] = kv + off
    return carry

  sems = (sem0, sem1, sem2, sem3)
  ngroups = _NROW // _GROUP

  waves = [[pltpu.make_async_copy(feat_hbm.at[idx_v.at[g * _GROUP + i]],
                                  pred_v.at[g * _GROUP + i], sems[g])
            for i in range(_GROUP)]
           for g in range(ngroups)]
  for g in range(ngroups):
    lax.fori_loop(g * _GROUP, (g + 1) * _GROUP, build, 0)
    for cp in waves[g]:
      cp.start()

  cp_tgt.wait()
  cp_mask.wait()

  def cbody(j, accs):
    a = list(accs)
    for h in range(2):
      cc = 2 * j + h
      for kg in range(_KG):
        pv = pred_v[j, pl.ds(h * 64 + kg * _L, _L)]
        tv = plsc.load_gather(tgt_v, [iota + kg * _L,
                                      jnp.zeros((_L,), jnp.int32) + cc])
        a[kg] = a[kg] + jnp.abs(pv - tv)
    return tuple(a)

  z = jnp.zeros((_L,), jnp.float32)
  accs = (z,) * _KG
  for g in range(ngroups):
    for cp in waves[g]:
      cp.wait()
    accs = lax.fori_loop(g * _GROUP, (g + 1) * _GROUP, cbody, accs,
                         unroll=2)

  num = jnp.float32(0.0)
  den = jnp.float32(0.0)
  for kg in range(_KG):
    mv = mask_v[pl.ds(kg * _L, _L)]
    num = num + jnp.sum(accs[kg] * jnp.abs(mv))
    den = den + jnp.sum(mv)

  out_v[...] = jnp.where(iota == 0, num, jnp.where(iota == 1, den, 0.0))
  pltpu.sync_copy(out_v, out_hbm.at[wid])


_SC_LOSS = pl.kernel(
    _body,
    out_type=jax.ShapeDtypeStruct((_NW, _L), jnp.float32),
    mesh=plsc.VectorSubcoreMesh(core_axis_name="c", subcore_axis_name="s"),
    compiler_params=pltpu.CompilerParams(needs_layout_passes=False,
                                         use_tc_tiling_on_sc=True),
    scratch_types=[
        pltpu.VMEM((_KCHUNK,), jnp.int32),
        pltpu.VMEM((_KCHUNK,), jnp.float32),
        pltpu.VMEM((_NROW, 128), jnp.int32),
        pltpu.VMEM((_KCHUNK, _C), jnp.float32),
        pltpu.VMEM((_NROW, 128), jnp.float32),
        pltpu.VMEM((_L,), jnp.float32),
        pltpu.SemaphoreType.DMA,
        pltpu.SemaphoreType.DMA,
        pltpu.SemaphoreType.DMA,
        pltpu.SemaphoreType.DMA,
        pltpu.SemaphoreType.DMA,
        pltpu.SemaphoreType.DMA,
    ],
)


@jax.jit
def kernel(output, mask, ind, target, freq_mask, hm):
  feat = output.reshape(_B * _C * _HW)
  parts = _SC_LOSS(feat, target, ind, mask)
  tot = jnp.sum(parts, axis=0)
  return tot[0] / (_C * tot[1] + 1e-4)


# speedup vs baseline: 1.0141x; 1.0067x over previous
import jax
import jax.numpy as jnp
from jax import lax
from jax.experimental import pallas as pl
from jax.experimental.pallas import tpu as pltpu
from jax.experimental.pallas import tpu_sc as plsc

_B, _C, _H, _W, _K = 16, 64, 128, 128, 128
_HW = _H * _W
_NC, _NS, _L = 2, 16, 16
_NW = _NC * _NS
_KCHUNK = (_B * _K) // _NW
_KG = _KCHUNK // _L
_NROW = (_KCHUNK * _C) // 128
_GROUP = 8


def _body(feat_hbm, tgt_hbm, ind_hbm, mask_hbm, out_hbm,
          ind_v, mask_v, idx_v, tgt_v, pred_v, out_v,
          sem0, sem1, sem2, sem3, sem_t, sem_i):
  wid = lax.axis_index("s") * _NC + lax.axis_index("c")
  b = wid // 2

  cp_ind = pltpu.make_async_copy(
      ind_hbm.at[pl.ds(wid * _KCHUNK, _KCHUNK)], ind_v, sem_i)
  cp_ind.start()
  cp_tgt = pltpu.make_async_copy(
      tgt_hbm.at[pl.ds(wid * _KCHUNK * _C, _KCHUNK * _C)], tgt_v, sem_t)
  cp_tgt.start()
  cp_mask = pltpu.make_async_copy(
      mask_hbm.at[pl.ds(wid * _KCHUNK, _KCHUNK)], mask_v, sem_t)
  cp_mask.start()
  cp_ind.wait()

  iota = lax.iota(jnp.int32, _L)
  base = b * _C * _HW

  def build(j, carry):
    for h in range(2):
      off = base + (2 * j + h) * _HW
      for kg in range(_KG):
        kv = ind_v[pl.ds(kg * _L, _L)]
        idx_v[j, pl.ds(h * 64 + kg * _L, _L)] = kv + off
    return carry

  sems = (sem0, sem1, sem2, sem3)
  ngroups = _NROW // _GROUP

  waves = [[pltpu.make_async_copy(feat_hbm.at[idx_v.at[g * _GROUP + i]],
                                  pred_v.at[g * _GROUP + i], sems[g])
            for i in range(_GROUP)]
           for g in range(ngroups)]
  for g in range(ngroups):
    lax.fori_loop(g * _GROUP, (g + 1) * _GROUP, build, 0)
    for cp in waves[g]:
      cp.start()

  cp_tgt.wait()
  cp_mask.wait()

  kbase = [(iota + kg * _L) * _C for kg in range(_KG)]

  def cbody(j, accs):
    a = list(accs)
    for h in range(2):
      cc = 2 * j + h
      for kg in range(_KG):
        pv = pred_v[j, pl.ds(h * 64 + kg * _L, _L)]
        tv = plsc.load_gather(tgt_v, [kbase[kg] + cc])
        a[kg] = a[kg] + jnp.abs(pv - tv)
    return tuple(a)

  z = jnp.zeros((_L,), jnp.float32)
  accs = (z,) * _KG
  for g in range(ngroups):
    for cp in waves[g]:
      cp.wait()
    accs = lax.fori_loop(g * _GROUP, (g + 1) * _GROUP, cbody, accs,
                         unroll=2)

  num = jnp.float32(0.0)
  den = jnp.float32(0.0)
  for kg in range(_KG):
    mv = mask_v[pl.ds(kg * _L, _L)]
    num = num + jnp.sum(accs[kg] * jnp.abs(mv))
    den = den + jnp.sum(mv)

  out_v[...] = jnp.where(iota == 0, num, jnp.where(iota == 1, den, 0.0))
  pltpu.sync_copy(out_v, out_hbm.at[wid])


_SC_LOSS = pl.kernel(
    _body,
    out_type=jax.ShapeDtypeStruct((_NW, _L), jnp.float32),
    mesh=plsc.VectorSubcoreMesh(core_axis_name="c", subcore_axis_name="s"),
    compiler_params=pltpu.CompilerParams(needs_layout_passes=False),
    scratch_types=[
        pltpu.VMEM((_KCHUNK,), jnp.int32),
        pltpu.VMEM((_KCHUNK,), jnp.float32),
        pltpu.VMEM((_NROW, 128), jnp.int32),
        pltpu.VMEM((_KCHUNK * _C,), jnp.float32),
        pltpu.VMEM((_NROW, 128), jnp.float32),
        pltpu.VMEM((_L,), jnp.float32),
        pltpu.SemaphoreType.DMA,
        pltpu.SemaphoreType.DMA,
        pltpu.SemaphoreType.DMA,
        pltpu.SemaphoreType.DMA,
        pltpu.SemaphoreType.DMA,
        pltpu.SemaphoreType.DMA,
    ],
)


@jax.jit
def kernel(output, mask, ind, target, freq_mask, hm):
  feat = output.reshape(_B * _C * _HW)
  tgt = target.reshape(_B * _K * _C)
  indf = ind.reshape(_B * _K)
  maskf = mask.reshape(_B * _K)
  parts = _SC_LOSS(feat, tgt, indf, maskf)
  tot = jnp.sum(parts, axis=0)
  return tot[0] / (_C * tot[1] + 1e-4)
